# baseline (speedup 1.0000x reference)
import functools
import math

import jax
import jax.numpy as jnp
from jax.experimental import pallas as pl


N_BLOCK = 4096


def _vmf_kernel(value_ref, locs_ref, log_scales_ref, logits_ref, out_ref, *, d):
    mu = locs_ref[...]
    norm = jnp.sqrt(jnp.sum(mu * mu, axis=1, keepdims=True))
    mu_p = mu / norm
    kappa = jnp.exp(log_scales_ref[...])

    v = d / 2.0 - 1.0
    z = kappa / v
    sq = jnp.sqrt(1.0 + z * z)
    eta = sq + jnp.log(z) - jnp.log(1.0 + sq)
    log_iv = v * eta - 0.5 * jnp.log(2.0 * jnp.pi * v) - 0.25 * jnp.log1p(z * z)
    log_ive = log_iv - kappa
    log_norm = -((d / 2.0 - 1.0) * jnp.log(kappa)
                 - (d / 2.0) * math.log(2.0 * math.pi)
                 - (kappa + log_ive))

    logits = logits_ref[...]
    w = jax.nn.softmax(logits, axis=1)
    bias = (jnp.log(w) - log_norm.T)

    x = value_ref[...]
    dots = jax.lax.dot_general(
        x, mu_p, (((1,), (1,)), ((), ())),
        preferred_element_type=jnp.float32)
    lp = kappa.T * dots + bias
    mx = jnp.max(lp, axis=1, keepdims=True)
    out = mx[:, 0] + jnp.log(jnp.sum(jnp.exp(lp - mx), axis=1))
    out_ref[...] = out


def kernel(value, locs, log_scales, mixture_weights_logits):
    n, d = value.shape
    m = locs.shape[0]
    logits2d = mixture_weights_logits.reshape(1, m)
    grid = (n // N_BLOCK,)
    return pl.pallas_call(
        functools.partial(_vmf_kernel, d=d),
        grid=grid,
        in_specs=[
            pl.BlockSpec((N_BLOCK, d), lambda i: (i, 0)),
            pl.BlockSpec((m, d), lambda i: (0, 0)),
            pl.BlockSpec((m, 1), lambda i: (0, 0)),
            pl.BlockSpec((1, m), lambda i: (0, 0)),
        ],
        out_specs=pl.BlockSpec((N_BLOCK,), lambda i: (i,)),
        out_shape=jax.ShapeDtypeStruct((n,), jnp.float32),
    )(value, locs, log_scales, logits2d)


# speedup vs baseline: 1.8459x; 1.8459x over previous
import functools
import math

import jax
import jax.numpy as jnp
from jax.experimental import pallas as pl


N_BLOCK = 4096


def _vmf_kernel(value_ref, locs_ref, log_scales_ref, logits_ref, out_ref, *, d):
    mu = locs_ref[...]
    norm = jnp.sqrt(jnp.sum(mu * mu, axis=1, keepdims=True))
    mu_p = mu / norm
    kappa = jnp.exp(log_scales_ref[...])

    v = d / 2.0 - 1.0
    z = kappa / v
    sq = jnp.sqrt(1.0 + z * z)
    eta = sq + jnp.log(z) - jnp.log(1.0 + sq)
    log_iv = v * eta - 0.5 * jnp.log(2.0 * jnp.pi * v) - 0.25 * jnp.log1p(z * z)
    log_ive = log_iv - kappa
    log_norm = -((d / 2.0 - 1.0) * jnp.log(kappa)
                 - (d / 2.0) * math.log(2.0 * math.pi)
                 - (kappa + log_ive))

    logits = logits_ref[...]
    w = jax.nn.softmax(logits, axis=0)
    bias = (jnp.log(w) - log_norm)

    x = value_ref[...]
    dots = jax.lax.dot_general(
        x, mu_p, (((1,), (1,)), ((), ())),
        preferred_element_type=jnp.float32)
    lp = kappa * dots.T + bias
    mx = jnp.max(lp, axis=0, keepdims=True)
    out = mx[0] + jnp.log(jnp.sum(jnp.exp(lp - mx), axis=0))
    out_ref[...] = out


def kernel(value, locs, log_scales, mixture_weights_logits):
    n, d = value.shape
    m = locs.shape[0]
    logits2d = mixture_weights_logits.reshape(m, 1)
    grid = (n // N_BLOCK,)
    return pl.pallas_call(
        functools.partial(_vmf_kernel, d=d),
        grid=grid,
        in_specs=[
            pl.BlockSpec((N_BLOCK, d), lambda i: (i, 0)),
            pl.BlockSpec((m, d), lambda i: (0, 0)),
            pl.BlockSpec((m, 1), lambda i: (0, 0)),
            pl.BlockSpec((m, 1), lambda i: (0, 0)),
        ],
        out_specs=pl.BlockSpec((N_BLOCK,), lambda i: (i,)),
        out_shape=jax.ShapeDtypeStruct((n,), jnp.float32),
    )(value, locs, log_scales, logits2d)


# speedup vs baseline: 1.8496x; 1.0020x over previous
import functools
import math

import jax
import jax.numpy as jnp
from jax.experimental import pallas as pl


N_BLOCK = 8192


def _vmf_kernel(value_ref, locs_ref, log_scales_ref, logits_ref, out_ref, *, d):
    mu = locs_ref[...]
    norm = jnp.sqrt(jnp.sum(mu * mu, axis=1, keepdims=True))
    mu_p = mu / norm
    kappa = jnp.exp(log_scales_ref[...])

    v = d / 2.0 - 1.0
    z = kappa / v
    sq = jnp.sqrt(1.0 + z * z)
    eta = sq + jnp.log(z) - jnp.log(1.0 + sq)
    log_iv = v * eta - 0.5 * jnp.log(2.0 * jnp.pi * v) - 0.25 * jnp.log1p(z * z)
    log_ive = log_iv - kappa
    log_norm = -((d / 2.0 - 1.0) * jnp.log(kappa)
                 - (d / 2.0) * math.log(2.0 * math.pi)
                 - (kappa + log_ive))

    logits = logits_ref[...]
    w = jax.nn.softmax(logits, axis=0)
    bias = (jnp.log(w) - log_norm)

    x = value_ref[...]
    dots = jax.lax.dot_general(
        x, mu_p, (((1,), (1,)), ((), ())),
        preferred_element_type=jnp.float32)
    lp = kappa * dots.T + bias
    mx = jnp.max(lp, axis=0, keepdims=True)
    out = mx[0] + jnp.log(jnp.sum(jnp.exp(lp - mx), axis=0))
    out_ref[...] = out


def kernel(value, locs, log_scales, mixture_weights_logits):
    n, d = value.shape
    m = locs.shape[0]
    logits2d = mixture_weights_logits.reshape(m, 1)
    grid = (n // N_BLOCK,)
    return pl.pallas_call(
        functools.partial(_vmf_kernel, d=d),
        grid=grid,
        in_specs=[
            pl.BlockSpec((N_BLOCK, d), lambda i: (i, 0)),
            pl.BlockSpec((m, d), lambda i: (0, 0)),
            pl.BlockSpec((m, 1), lambda i: (0, 0)),
            pl.BlockSpec((m, 1), lambda i: (0, 0)),
        ],
        out_specs=pl.BlockSpec((N_BLOCK,), lambda i: (i,)),
        out_shape=jax.ShapeDtypeStruct((n,), jnp.float32),
    )(value, locs, log_scales, logits2d)
